# baseline (speedup 1.0000x reference)
import functools
import jax
import jax.numpy as jnp
from jax import lax
from jax.experimental import pallas as pl
from jax.experimental.pallas import tpu as pltpu
from jax.experimental.pallas import tpu_sc as plsc

_E = 8
_R = 16
_SCALING = 2.0
_TN = 512
_TNR = 1024
_CH = 128
_NW = 32
_L = 16
_H = 1024


def _router_kernel(x_ref, wr_ref, aall_ref, wb1_ref, out_ref, xa_ref, b1_ref):
    x = x_ref[...]
    for b in range(_TNR // _CH):
        out_ref[b] = lax.dot_general(
            wr_ref[...], x[b * _CH:(b + 1) * _CH], (((1,), (1,)), ((), ())),
            preferred_element_type=jnp.float32)
    xa_ref[...] = lax.dot_general(
        x, aall_ref[...], (((1,), (1,)), ((), ())),
        preferred_element_type=jnp.float32)
    b1_ref[...] = lax.dot_general(
        x, wb1_ref[...], (((1,), (1,)), ((), ())),
        preferred_element_type=jnp.float32)


def _sc_gate_body(logits_hbm, gate_hbm, buf_in, buf_out):
    wid = lax.axis_index("s") * 2 + lax.axis_index("c")
    pltpu.sync_copy(logits_hbm.at[wid], buf_in)
    for g in range(_CH // _L):
        sl = pl.ds(g * _L, _L)
        vs = [buf_in[e, sl] for e in range(_E)]
        m1 = vs[0]
        for v in vs[1:]:
            m1 = jnp.maximum(m1, v)
        neg = jnp.full((_L,), -jnp.inf, jnp.float32)
        m2 = neg
        for v in vs:
            m2 = jnp.maximum(m2, jnp.where(v < m1, v, neg))
        ws = [jnp.where(v >= m2, jnp.exp(v - m1), 0.0) for v in vs]
        s = ws[0]
        for w in ws[1:]:
            s = s + w
        inv = 1.0 / s
        for e in range(_E):
            buf_out[e, sl] = ws[e] * inv
    pltpu.sync_copy(buf_out, gate_hbm.at[wid])


def _sc_gate(logits3):
    mesh = plsc.VectorSubcoreMesh(core_axis_name="c", subcore_axis_name="s")
    fn = functools.partial(
        pl.kernel,
        mesh=mesh,
        out_type=jax.ShapeDtypeStruct((_NW, _E, _CH), jnp.float32),
        scratch_types=[
            pltpu.VMEM((_E, _CH), jnp.float32),
            pltpu.VMEM((_E, _CH), jnp.float32),
        ],
    )(_sc_gate_body)
    return fn(logits3)


def _main_kernel(x_ref, wb2_ref, xa_ref, ball_ref, expand_ref, bbase_ref,
                 gate_ref, b1_ref, out_ref):
    x = x_ref[...]
    subscales = []
    for b in range(_TN // _CH):
        subscales.append(lax.dot_general(
            gate_ref[b], expand_ref[...], (((0,), (0,)), ((), ())),
            preferred_element_type=jnp.float32))
    scale = jnp.concatenate(subscales, axis=0)
    xa = xa_ref[...] * scale
    lora = lax.dot_general(
        xa, ball_ref[...], (((1,), (1,)), ((), ())),
        preferred_element_type=jnp.float32)
    base2 = lax.dot_general(
        x, wb2_ref[...], (((1,), (1,)), ((), ())),
        preferred_element_type=jnp.float32)
    bias = bbase_ref[...]
    out_ref[:, :_H] = b1_ref[...] + lora[:, :_H] + bias[:, :_H]
    out_ref[:, _H:] = base2 + lora[:, _H:] + bias[:, _H:]


def kernel(x, W_base, b_base, W_router, A, B):
    bs, seq, d = x.shape
    out_dim = W_base.shape[0]
    n = bs * seq
    x2 = x.reshape(n, d)
    a_all = A.reshape(_E * _R, d)
    b_all = jnp.transpose(B, (1, 0, 2)).reshape(out_dim, _E * _R)
    bias = b_base.reshape(1, out_dim)
    expand = jnp.kron(jnp.eye(_E, dtype=jnp.float32),
                      jnp.ones((1, _R), jnp.float32)) * _SCALING

    nblkr = _TNR // _CH
    nblk = _TN // _CH
    logits3, xa, base1 = pl.pallas_call(
        _router_kernel,
        grid=(n // _TNR,),
        in_specs=[
            pl.BlockSpec((_TNR, d), lambda i: (i, 0)),
            pl.BlockSpec((_E, d), lambda i: (0, 0)),
            pl.BlockSpec((_E * _R, d), lambda i: (0, 0)),
            pl.BlockSpec((_H, d), lambda i: (0, 0)),
        ],
        out_specs=[
            pl.BlockSpec((nblkr, _E, _CH), lambda i: (i, 0, 0)),
            pl.BlockSpec((_TNR, _E * _R), lambda i: (i, 0)),
            pl.BlockSpec((_TNR, _H), lambda i: (i, 0)),
        ],
        out_shape=[
            jax.ShapeDtypeStruct((_NW, _E, _CH), jnp.float32),
            jax.ShapeDtypeStruct((n, _E * _R), jnp.float32),
            jax.ShapeDtypeStruct((n, _H), jnp.float32),
        ],
        compiler_params=pltpu.CompilerParams(
            dimension_semantics=("arbitrary",),
        ),
    )(x2, W_router, a_all, W_base)

    gate3 = _sc_gate(logits3)

    out = pl.pallas_call(
        _main_kernel,
        grid=(n // _TN,),
        in_specs=[
            pl.BlockSpec((_TN, d), lambda i: (i, 0)),
            pl.BlockSpec((out_dim - _H, d), lambda i: (1, 0)),
            pl.BlockSpec((_TN, _E * _R), lambda i: (i, 0)),
            pl.BlockSpec((out_dim, _E * _R), lambda i: (0, 0)),
            pl.BlockSpec((_E, _E * _R), lambda i: (0, 0)),
            pl.BlockSpec((1, out_dim), lambda i: (0, 0)),
            pl.BlockSpec((nblk, _E, _CH), lambda i: (i, 0, 0)),
            pl.BlockSpec((_TN, _H), lambda i: (i, 0)),
        ],
        out_specs=pl.BlockSpec((_TN, out_dim), lambda i: (i, 0)),
        out_shape=jax.ShapeDtypeStruct((n, out_dim), jnp.float32),
        compiler_params=pltpu.CompilerParams(
            dimension_semantics=("arbitrary",),
        ),
    )(x2, W_base, xa, b_all, expand, bias, gate3, base1)
    return out.reshape(bs, seq, out_dim)


# speedup vs baseline: 1.0576x; 1.0576x over previous
import functools
import jax
import jax.numpy as jnp
from jax import lax
from jax.experimental import pallas as pl
from jax.experimental.pallas import tpu as pltpu
from jax.experimental.pallas import tpu_sc as plsc

_E = 8
_R = 16
_SCALING = 2.0
_TN = 512
_TNR = 1024
_CH = 128
_NW = 32
_L = 16


def _router_kernel(x_ref, wr_ref, aall_ref, out_ref, xa_ref):
    x = x_ref[...]
    for b in range(_TNR // _CH):
        out_ref[b] = lax.dot_general(
            wr_ref[...], x[b * _CH:(b + 1) * _CH], (((1,), (1,)), ((), ())),
            preferred_element_type=jnp.float32)
    xa_ref[...] = lax.dot_general(
        x, aall_ref[...], (((1,), (1,)), ((), ())),
        preferred_element_type=jnp.float32)


def _sc_gate_body(logits_hbm, gate_hbm, buf_in, buf_out):
    wid = lax.axis_index("s") * 2 + lax.axis_index("c")
    pltpu.sync_copy(logits_hbm.at[wid], buf_in)
    for g in range(_CH // _L):
        sl = pl.ds(g * _L, _L)
        vs = [buf_in[e, sl] for e in range(_E)]
        m1 = vs[0]
        for v in vs[1:]:
            m1 = jnp.maximum(m1, v)
        neg = jnp.full((_L,), -jnp.inf, jnp.float32)
        m2 = neg
        for v in vs:
            m2 = jnp.maximum(m2, jnp.where(v < m1, v, neg))
        ws = [jnp.where(v >= m2, jnp.exp(v - m1), 0.0) for v in vs]
        s = ws[0]
        for w in ws[1:]:
            s = s + w
        inv = 1.0 / s
        for e in range(_E):
            buf_out[e, sl] = ws[e] * inv
    pltpu.sync_copy(buf_out, gate_hbm.at[wid])


def _sc_gate(logits3):
    mesh = plsc.VectorSubcoreMesh(core_axis_name="c", subcore_axis_name="s")
    fn = functools.partial(
        pl.kernel,
        mesh=mesh,
        out_type=jax.ShapeDtypeStruct((_NW, _E, _CH), jnp.float32),
        scratch_types=[
            pltpu.VMEM((_E, _CH), jnp.float32),
            pltpu.VMEM((_E, _CH), jnp.float32),
        ],
    )(_sc_gate_body)
    return fn(logits3)


def _main_kernel(x_ref, wb_ref, xa_ref, ball_ref, expand_ref, bbase_ref,
                 gate_ref, out_ref):
    x = x_ref[...]
    subscales = []
    for b in range(_TN // _CH):
        subscales.append(lax.dot_general(
            gate_ref[b], expand_ref[...], (((0,), (0,)), ((), ())),
            preferred_element_type=jnp.float32))
    scale = jnp.concatenate(subscales, axis=0)
    xa = xa_ref[...] * scale
    lora = lax.dot_general(
        xa, ball_ref[...], (((1,), (1,)), ((), ())),
        preferred_element_type=jnp.float32)
    base = lax.dot_general(
        x, wb_ref[...], (((1,), (1,)), ((), ())),
        preferred_element_type=jnp.float32)
    out_ref[...] = base + lora + bbase_ref[...]


def kernel(x, W_base, b_base, W_router, A, B):
    bs, seq, d = x.shape
    out_dim = W_base.shape[0]
    n = bs * seq
    x2 = x.reshape(n, d)
    a_all = A.reshape(_E * _R, d)
    b_all = jnp.transpose(B, (1, 0, 2)).reshape(out_dim, _E * _R)
    bias = b_base.reshape(1, out_dim)
    expand = jnp.kron(jnp.eye(_E, dtype=jnp.float32),
                      jnp.ones((1, _R), jnp.float32)) * _SCALING

    nblkr = _TNR // _CH
    nblk = _TN // _CH
    logits3, xa = pl.pallas_call(
        _router_kernel,
        grid=(n // _TNR,),
        in_specs=[
            pl.BlockSpec((_TNR, d), lambda i: (i, 0)),
            pl.BlockSpec((_E, d), lambda i: (0, 0)),
            pl.BlockSpec((_E * _R, d), lambda i: (0, 0)),
        ],
        out_specs=[
            pl.BlockSpec((nblkr, _E, _CH), lambda i: (i, 0, 0)),
            pl.BlockSpec((_TNR, _E * _R), lambda i: (i, 0)),
        ],
        out_shape=[
            jax.ShapeDtypeStruct((_NW, _E, _CH), jnp.float32),
            jax.ShapeDtypeStruct((n, _E * _R), jnp.float32),
        ],
        compiler_params=pltpu.CompilerParams(
            dimension_semantics=("arbitrary",),
        ),
    )(x2, W_router, a_all)

    gate3 = _sc_gate(logits3)

    out = pl.pallas_call(
        _main_kernel,
        grid=(n // _TN,),
        in_specs=[
            pl.BlockSpec((_TN, d), lambda i: (i, 0)),
            pl.BlockSpec((out_dim, d), lambda i: (0, 0)),
            pl.BlockSpec((_TN, _E * _R), lambda i: (i, 0)),
            pl.BlockSpec((out_dim, _E * _R), lambda i: (0, 0)),
            pl.BlockSpec((_E, _E * _R), lambda i: (0, 0)),
            pl.BlockSpec((1, out_dim), lambda i: (0, 0)),
            pl.BlockSpec((nblk, _E, _CH), lambda i: (i, 0, 0)),
        ],
        out_specs=pl.BlockSpec((_TN, out_dim), lambda i: (i, 0)),
        out_shape=jax.ShapeDtypeStruct((n, out_dim), jnp.float32),
        compiler_params=pltpu.CompilerParams(
            dimension_semantics=("arbitrary",),
        ),
    )(x2, W_base, xa, b_all, expand, bias, gate3)
    return out.reshape(bs, seq, out_dim)
